# baseline (speedup 1.0000x reference)
import functools

import jax
import jax.numpy as jnp
import numpy as np
from jax import lax
from jax.experimental import pallas as pl
from jax.experimental.pallas import tpu as pltpu
from jax.experimental.pallas import tpu_sc as plsc

N_NODES = 100000
DEG = 64
DEGP = 128
BATCH = 16384
S = 32

_info = plsc.get_sparse_core_info()
NC, NS, L = _info.num_cores, _info.num_subcores, _info.num_lanes
NW = NC * NS
B_PER_W = BATCH // NW
CHUNK = 128
CHUNKP = CHUNK + 1
NCHUNK = B_PER_W // CHUNK
NBAND = S // 8
NBLK = BATCH // CHUNK * NBAND

_PERM = np.array([
    35, 45, 31, 63, 7, 4, 29, 44, 16, 58, 37, 19, 61, 2, 34, 5,
    30, 42, 3, 39, 56, 22, 6, 54, 18, 10, 11, 53, 32, 15, 49, 50,
    20, 43, 8, 24, 9, 40, 59, 25, 13, 52, 62, 60, 47, 33, 14, 17,
    38, 23, 0, 41, 21, 26, 57, 1, 28, 48, 36, 55, 51, 27, 12, 46,
], dtype=np.int32)

_mesh = plsc.VectorSubcoreMesh(core_axis_name="c", subcore_axis_name="s")


@functools.partial(
    pl.kernel,
    mesh=_mesh,
    compiler_params=pltpu.CompilerParams(
        needs_layout_passes=False, use_tc_tiling_on_sc=False),
    out_type=jax.ShapeDtypeStruct((NBLK, 8, CHUNK), jnp.float32),
    scratch_types=[
        pltpu.VMEM((NCHUNK, CHUNK), jnp.int32),
        pltpu.VMEM((NCHUNK, CHUNK), jnp.int32),
        pltpu.VMEM((CHUNK, DEG), jnp.float32),
        pltpu.VMEM((CHUNK, DEG), jnp.float32),
        pltpu.VMEM((CHUNK, DEG), jnp.float32),
        pltpu.VMEM((CHUNK, DEG), jnp.float32),
        pltpu.VMEM((S, CHUNKP), jnp.float32),
        pltpu.VMEM((S, CHUNKP), jnp.float32),
        pltpu.VMEM((S,), jnp.int32),
        pltpu.VMEM((S,), jnp.float32),
        pltpu.SemaphoreType.DMA,
        pltpu.SemaphoreType.DMA,
        pltpu.SemaphoreType.DMA,
        pltpu.SemaphoreType.DMA,
        pltpu.SemaphoreType.DMA,
        pltpu.SemaphoreType.DMA,
    ],
)
def _sample_kernel(adj_hbm, ids_hbm, cols_hbm, mask_hbm, out_hbm,
                   idx_v, qidx_v, rows0_v, rows1_v, rows2_v, rows3_v,
                   outc0_v, outc1_v, cols_v, mask_v,
                   gsem0, gsem1, gsem2, gsem3, osem0, osem1):
    wid = lax.axis_index("s") * NC + lax.axis_index("c")

    pltpu.sync_copy(ids_hbm.at[pl.ds(wid * NCHUNK, NCHUNK)], idx_v)
    pltpu.sync_copy(cols_hbm, cols_v)
    pltpu.sync_copy(mask_hbm, mask_v)
    for c in range(NCHUNK):
        for g in range(CHUNK // L):
            nv = idx_v[c, pl.ds(g * L, L)]
            qidx_v[c, pl.ds(g * L, L)] = lax.shift_left(nv, 1)

    rows_bufs = (rows0_v, rows1_v, rows2_v, rows3_v)
    gsems = (gsem0, gsem1, gsem2, gsem3)
    out_bufs = (outc0_v, outc1_v)
    osems = (osem0, osem1)

    gcopies = [
        pltpu.async_copy(adj_hbm.at[qidx_v.at[c]], rows_bufs[c], gsems[c])
        for c in range(NCHUNK)
    ]

    cols_lo = cols_v[pl.ds(0, L)]
    cols_hi = cols_v[pl.ds(L, L)]
    m_lo = mask_v[pl.ds(0, L)]
    m_hi = mask_v[pl.ds(L, L)]
    j_lo = lax.iota(jnp.int32, L)
    j_hi = j_lo + L

    ocopies = [[], []]
    for c in range(NCHUNK):
        rows_v = rows_bufs[c]
        outc_v = out_bufs[c % 2]
        gcopies[c].wait()
        for cp in ocopies[c % 2]:
            cp.wait()

        @plsc.parallel_loop(0, CHUNK, unroll=8)
        def body(r):
            ridx = jnp.full((L,), r, dtype=jnp.int32)
            lo = plsc.load_gather(rows_v, [ridx, cols_lo]) * m_lo
            hi = plsc.load_gather(rows_v, [ridx, cols_hi]) * m_hi
            plsc.store_scatter(outc_v, [j_lo, ridx], lo)
            plsc.store_scatter(outc_v, [j_hi, ridx], hi)

        tile_c = wid * NCHUNK + c
        ocopies[c % 2] = [
            pltpu.async_copy(
                outc_v.at[pl.ds(tr * 8, 8), pl.ds(0, CHUNK)],
                out_hbm.at[tr * (BATCH // CHUNK) + tile_c],
                osems[c % 2])
            for tr in range(NBAND)
        ]

    for cps in ocopies:
        for cp in cps:
            cp.wait()


def kernel(adj_info, ids, num_samples, layer):
    del layer
    adj_padT = jnp.pad(adj_info.T, ((0, DEGP - DEG), (0, 0)))
    adj2 = adj_padT.T.reshape(2 * N_NODES, DEG)
    cols = jnp.asarray(_PERM[:S])
    mask = (jnp.arange(S) < num_samples).astype(jnp.float32)
    ids2d = ids.reshape(NW * NCHUNK, CHUNK)
    out3 = _sample_kernel(adj2, ids2d, cols, mask)
    out = out3.reshape(NBAND, BATCH // CHUNK, 8, CHUNK)
    return out.transpose(1, 3, 0, 2).reshape(BATCH, S)


# speedup vs baseline: 1.1467x; 1.1467x over previous
import functools

import jax
import jax.numpy as jnp
import numpy as np
from jax import lax
from jax.experimental import pallas as pl
from jax.experimental.pallas import tpu as pltpu
from jax.experimental.pallas import tpu_sc as plsc

N_NODES = 100000
DEG = 64
DEGP = 128
BATCH = 16384
S = 32

_info = plsc.get_sparse_core_info()
NC, NS, L = _info.num_cores, _info.num_subcores, _info.num_lanes
NW = NC * NS
B_PER_W = BATCH // NW
CHUNK = 128
CHUNKP = CHUNK + 1
NCHUNK = B_PER_W // CHUNK
NBAND = S // 8
NBLK = BATCH // CHUNK * NBAND

_PERM = np.array([
    35, 45, 31, 63, 7, 4, 29, 44, 16, 58, 37, 19, 61, 2, 34, 5,
    30, 42, 3, 39, 56, 22, 6, 54, 18, 10, 11, 53, 32, 15, 49, 50,
    20, 43, 8, 24, 9, 40, 59, 25, 13, 52, 62, 60, 47, 33, 14, 17,
    38, 23, 0, 41, 21, 26, 57, 1, 28, 48, 36, 55, 51, 27, 12, 46,
], dtype=np.int32)

_mesh = plsc.VectorSubcoreMesh(core_axis_name="c", subcore_axis_name="s")


@functools.partial(
    pl.kernel,
    mesh=_mesh,
    compiler_params=pltpu.CompilerParams(
        needs_layout_passes=False, use_tc_tiling_on_sc=False),
    out_type=jax.ShapeDtypeStruct((NBLK, 8, CHUNK), jnp.float32),
    scratch_types=[
        pltpu.VMEM((NCHUNK, CHUNK), jnp.int32),
        pltpu.VMEM((NCHUNK, CHUNK), jnp.int32),
        pltpu.VMEM((CHUNK, DEG), jnp.float32),
        pltpu.VMEM((CHUNK, DEG), jnp.float32),
        pltpu.VMEM((CHUNK, DEG), jnp.float32),
        pltpu.VMEM((CHUNK, DEG), jnp.float32),
        pltpu.VMEM((S, CHUNKP), jnp.float32),
        pltpu.VMEM((S, CHUNKP), jnp.float32),
        pltpu.VMEM((S,), jnp.int32),
        pltpu.VMEM((S,), jnp.float32),
        pltpu.SemaphoreType.DMA,
        pltpu.SemaphoreType.DMA,
        pltpu.SemaphoreType.DMA,
        pltpu.SemaphoreType.DMA,
        pltpu.SemaphoreType.DMA,
        pltpu.SemaphoreType.DMA,
    ],
)
def _sample_kernel(adj_hbm, ids_hbm, cols_hbm, mask_hbm, out_hbm,
                   idx_v, qidx_v, rows0_v, rows1_v, rows2_v, rows3_v,
                   outc0_v, outc1_v, cols_v, mask_v,
                   gsem0, gsem1, gsem2, gsem3, osem0, osem1):
    wid = lax.axis_index("s") * NC + lax.axis_index("c")

    pltpu.sync_copy(ids_hbm.at[pl.ds(wid * NCHUNK, NCHUNK)], idx_v)
    pltpu.sync_copy(cols_hbm, cols_v)
    pltpu.sync_copy(mask_hbm, mask_v)
    for c in range(NCHUNK):
        for g in range(CHUNK // L):
            nv = idx_v[c, pl.ds(g * L, L)]
            qidx_v[c, pl.ds(g * L, L)] = lax.shift_left(nv, 1)

    rows_bufs = (rows0_v, rows1_v, rows2_v, rows3_v)
    gsems = (gsem0, gsem1, gsem2, gsem3)
    out_bufs = (outc0_v, outc1_v)
    osems = (osem0, osem1)

    gcopies = [
        pltpu.async_copy(adj_hbm.at[qidx_v.at[c]], rows_bufs[c], gsems[c])
        for c in range(NCHUNK)
    ]

    cols_lo = cols_v[pl.ds(0, L)]
    cols_hi = cols_v[pl.ds(L, L)]
    m_lo = mask_v[pl.ds(0, L)]
    m_hi = mask_v[pl.ds(L, L)]
    j_lo = lax.iota(jnp.int32, L)
    j_hi = j_lo + L

    ocopies = [[], []]
    for c in range(NCHUNK):
        rows_v = rows_bufs[c]
        outc_v = out_bufs[c % 2]
        gcopies[c].wait()
        for cp in ocopies[c % 2]:
            cp.wait()

        @plsc.parallel_loop(0, CHUNK, unroll=8)
        def body(r):
            ridx = jnp.full((L,), r, dtype=jnp.int32)
            lo = plsc.load_gather(rows_v, [ridx, cols_lo]) * m_lo
            hi = plsc.load_gather(rows_v, [ridx, cols_hi]) * m_hi
            plsc.store_scatter(outc_v, [j_lo, ridx], lo)
            plsc.store_scatter(outc_v, [j_hi, ridx], hi)

        tile_c = wid * NCHUNK + c
        ocopies[c % 2] = [
            pltpu.async_copy(
                outc_v.at[pl.ds(tr * 8, 8), pl.ds(0, CHUNK)],
                out_hbm.at[tr * (BATCH // CHUNK) + tile_c],
                osems[c % 2])
            for tr in range(NBAND)
        ]

    for cps in ocopies:
        for cp in cps:
            cp.wait()


def kernel(adj_info, ids, num_samples, layer):
    del layer
    adj_pad = jnp.pad(adj_info, ((0, 0), (0, DEGP - DEG)))
    adj2 = adj_pad.reshape(2 * N_NODES, DEG)
    cols = jnp.asarray(_PERM[:S])
    mask = (jnp.arange(S) < num_samples).astype(jnp.float32)
    ids2d = ids.reshape(NW * NCHUNK, CHUNK)
    out3 = _sample_kernel(adj2, ids2d, cols, mask)
    out = out3.reshape(NBAND, BATCH // CHUNK, 8, CHUNK)
    return out.transpose(1, 3, 0, 2).reshape(BATCH, S)


# speedup vs baseline: 1.2494x; 1.0896x over previous
import functools

import jax
import jax.numpy as jnp
import numpy as np
from jax import lax
from jax.experimental import pallas as pl
from jax.experimental.pallas import tpu as pltpu
from jax.experimental.pallas import tpu_sc as plsc

N_NODES = 100000
DEG = 64
DEGP = 128
BATCH = 16384
S = 32

_info = plsc.get_sparse_core_info()
NC, NS, L = _info.num_cores, _info.num_subcores, _info.num_lanes
NW = NC * NS
B_PER_W = BATCH // NW
CHUNK = 128
CHUNKP = CHUNK + 1
NCHUNK = B_PER_W // CHUNK
NBAND = S // 8
NBLK = BATCH // CHUNK * NBAND

_PERM = np.array([
    35, 45, 31, 63, 7, 4, 29, 44, 16, 58, 37, 19, 61, 2, 34, 5,
    30, 42, 3, 39, 56, 22, 6, 54, 18, 10, 11, 53, 32, 15, 49, 50,
    20, 43, 8, 24, 9, 40, 59, 25, 13, 52, 62, 60, 47, 33, 14, 17,
    38, 23, 0, 41, 21, 26, 57, 1, 28, 48, 36, 55, 51, 27, 12, 46,
], dtype=np.int32)

_mesh = plsc.VectorSubcoreMesh(core_axis_name="c", subcore_axis_name="s")


@functools.partial(
    pl.kernel,
    mesh=_mesh,
    compiler_params=pltpu.CompilerParams(
        needs_layout_passes=False, use_tc_tiling_on_sc=False),
    out_type=jax.ShapeDtypeStruct((NBLK, 8, CHUNK), jnp.float32),
    scratch_types=[
        pltpu.VMEM((NCHUNK, CHUNK), jnp.int32),
        pltpu.VMEM((NCHUNK, CHUNK), jnp.int32),
        pltpu.VMEM((CHUNK, DEG), jnp.float32),
        pltpu.VMEM((CHUNK, DEG), jnp.float32),
        pltpu.VMEM((CHUNK, DEG), jnp.float32),
        pltpu.VMEM((CHUNK, DEG), jnp.float32),
        pltpu.VMEM((S, CHUNKP), jnp.float32),
        pltpu.VMEM((S, CHUNKP), jnp.float32),
        pltpu.VMEM((S,), jnp.int32),
        pltpu.VMEM((S,), jnp.float32),
        pltpu.SemaphoreType.DMA,
        pltpu.SemaphoreType.DMA,
        pltpu.SemaphoreType.DMA,
        pltpu.SemaphoreType.DMA,
        pltpu.SemaphoreType.DMA,
        pltpu.SemaphoreType.DMA,
    ],
)
def _sample_kernel(adj_hbm, ids_hbm, cols_hbm, mask_hbm, out_hbm,
                   idx_v, qidx_v, rows0_v, rows1_v, rows2_v, rows3_v,
                   outc0_v, outc1_v, cols_v, mask_v,
                   gsem0, gsem1, gsem2, gsem3, osem0, osem1):
    wid = lax.axis_index("s") * NC + lax.axis_index("c")

    pltpu.sync_copy(ids_hbm.at[pl.ds(wid * NCHUNK, NCHUNK)], idx_v)
    pltpu.sync_copy(cols_hbm, cols_v)
    pltpu.sync_copy(mask_hbm, mask_v)
    for c in range(NCHUNK):
        for g in range(CHUNK // L):
            nv = idx_v[c, pl.ds(g * L, L)]
            qidx_v[c, pl.ds(g * L, L)] = lax.shift_left(nv, 1)

    rows_bufs = (rows0_v, rows1_v, rows2_v, rows3_v)
    gsems = (gsem0, gsem1, gsem2, gsem3)
    out_bufs = (outc0_v, outc1_v)
    osems = (osem0, osem1)

    gcopies = [
        pltpu.async_copy(adj_hbm.at[qidx_v.at[c]], rows_bufs[c], gsems[c])
        for c in range(NCHUNK)
    ]

    cols_lo = cols_v[pl.ds(0, L)]
    cols_hi = cols_v[pl.ds(L, L)]
    m_lo = mask_v[pl.ds(0, L)]
    m_hi = mask_v[pl.ds(L, L)]
    j_lo = lax.iota(jnp.int32, L)
    j_hi = j_lo + L

    ocopies = [[], []]
    for c in range(NCHUNK):
        rows_v = rows_bufs[c]
        outc_v = out_bufs[c % 2]
        gcopies[c].wait()
        for cp in ocopies[c % 2]:
            cp.wait()

        @plsc.parallel_loop(0, CHUNK, unroll=8)
        def body(r):
            ridx = jnp.full((L,), r, dtype=jnp.int32)
            lo = plsc.load_gather(rows_v, [ridx, cols_lo]) * m_lo
            hi = plsc.load_gather(rows_v, [ridx, cols_hi]) * m_hi
            plsc.store_scatter(outc_v, [j_lo, ridx], lo)
            plsc.store_scatter(outc_v, [j_hi, ridx], hi)

        tile_c = wid * NCHUNK + c
        ocopies[c % 2] = [
            pltpu.async_copy(
                outc_v.at[pl.ds(tr * 8, 8), pl.ds(0, CHUNK)],
                out_hbm.at[tr * (BATCH // CHUNK) + tile_c],
                osems[c % 2])
            for tr in range(NBAND)
        ]

    for cps in ocopies:
        for cp in cps:
            cp.wait()


TBLK = 2048
NTBLK = (N_NODES + TBLK - 1) // TBLK


def _transpose_pad_body(i_ref, o_ref):
    o_ref[...] = jnp.pad(i_ref[...].T, ((0, 0), (0, DEGP - DEG)))


_transpose_pad = pl.pallas_call(
    _transpose_pad_body,
    grid=(NTBLK,),
    in_specs=[pl.BlockSpec((DEG, TBLK), lambda i: (0, i))],
    out_specs=pl.BlockSpec((TBLK, DEGP), lambda i: (i, 0)),
    out_shape=jax.ShapeDtypeStruct((N_NODES, DEGP), jnp.float32),
)


def kernel(adj_info, ids, num_samples, layer):
    del layer
    adj_pad = _transpose_pad(adj_info.T)
    adj2 = adj_pad.reshape(2 * N_NODES, DEG)
    cols = jnp.asarray(_PERM[:S])
    mask = (jnp.arange(S) < num_samples).astype(jnp.float32)
    ids2d = ids.reshape(NW * NCHUNK, CHUNK)
    out3 = _sample_kernel(adj2, ids2d, cols, mask)
    out = out3.reshape(NBAND, BATCH // CHUNK, 8, CHUNK)
    return out.transpose(1, 3, 0, 2).reshape(BATCH, S)


# speedup vs baseline: 1.5259x; 1.2213x over previous
import functools

import jax
import jax.numpy as jnp
import numpy as np
from jax import lax
from jax.experimental import pallas as pl
from jax.experimental.pallas import tpu as pltpu
from jax.experimental.pallas import tpu_sc as plsc

N_NODES = 100000
DEG = 64
DEGP = 128
BATCH = 16384
S = 32

_info = plsc.get_sparse_core_info()
NC, NS, L = _info.num_cores, _info.num_subcores, _info.num_lanes
NW = NC * NS
B_PER_W = BATCH // NW
CHUNK = 128
CHUNKP = CHUNK + 1
NCHUNK = B_PER_W // CHUNK
NBAND = S // 8
NBLK = BATCH // CHUNK * NBAND

_PERM = np.array([
    35, 45, 31, 63, 7, 4, 29, 44, 16, 58, 37, 19, 61, 2, 34, 5,
    30, 42, 3, 39, 56, 22, 6, 54, 18, 10, 11, 53, 32, 15, 49, 50,
    20, 43, 8, 24, 9, 40, 59, 25, 13, 52, 62, 60, 47, 33, 14, 17,
    38, 23, 0, 41, 21, 26, 57, 1, 28, 48, 36, 55, 51, 27, 12, 46,
], dtype=np.int32)

_mesh = plsc.VectorSubcoreMesh(core_axis_name="c", subcore_axis_name="s")


@functools.partial(
    pl.kernel,
    mesh=_mesh,
    compiler_params=pltpu.CompilerParams(
        needs_layout_passes=False, use_tc_tiling_on_sc=False),
    out_type=jax.ShapeDtypeStruct((NBLK, 8, CHUNK), jnp.float32),
    scratch_types=[
        pltpu.VMEM((NCHUNK, CHUNK), jnp.int32),
        pltpu.VMEM((NCHUNK, CHUNK), jnp.int32),
        pltpu.VMEM((CHUNK, DEG), jnp.float32),
        pltpu.VMEM((CHUNK, DEG), jnp.float32),
        pltpu.VMEM((CHUNK, DEG), jnp.float32),
        pltpu.VMEM((CHUNK, DEG), jnp.float32),
        pltpu.VMEM((S, CHUNKP), jnp.float32),
        pltpu.VMEM((S, CHUNKP), jnp.float32),
        pltpu.VMEM((S,), jnp.int32),
        pltpu.VMEM((S,), jnp.float32),
        pltpu.SemaphoreType.DMA,
        pltpu.SemaphoreType.DMA,
        pltpu.SemaphoreType.DMA,
        pltpu.SemaphoreType.DMA,
        pltpu.SemaphoreType.DMA,
        pltpu.SemaphoreType.DMA,
    ],
)
def _sample_kernel(adj_hbm, ids_hbm, cols_hbm, mask_hbm, out_hbm,
                   idx_v, qidx_v, rows0_v, rows1_v, rows2_v, rows3_v,
                   outc0_v, outc1_v, cols_v, mask_v,
                   gsem0, gsem1, gsem2, gsem3, osem0, osem1):
    wid = lax.axis_index("s") * NC + lax.axis_index("c")

    pltpu.sync_copy(ids_hbm.at[pl.ds(wid * NCHUNK, NCHUNK)], idx_v)
    pltpu.sync_copy(cols_hbm, cols_v)
    pltpu.sync_copy(mask_hbm, mask_v)
    for c in range(NCHUNK):
        for g in range(CHUNK // L):
            nv = idx_v[c, pl.ds(g * L, L)]
            qidx_v[c, pl.ds(g * L, L)] = lax.shift_left(nv, 1)

    rows_bufs = (rows0_v, rows1_v, rows2_v, rows3_v)
    gsems = (gsem0, gsem1, gsem2, gsem3)
    out_bufs = (outc0_v, outc1_v)
    osems = (osem0, osem1)

    gcopies = [
        pltpu.async_copy(adj_hbm.at[qidx_v.at[c]], rows_bufs[c], gsems[c])
        for c in range(NCHUNK)
    ]

    cols_lo = cols_v[pl.ds(0, L)]
    cols_hi = cols_v[pl.ds(L, L)]
    m_lo = mask_v[pl.ds(0, L)]
    m_hi = mask_v[pl.ds(L, L)]
    j_lo = lax.iota(jnp.int32, L)
    j_hi = j_lo + L

    ocopies = [[], []]
    for c in range(NCHUNK):
        rows_v = rows_bufs[c]
        outc_v = out_bufs[c % 2]
        gcopies[c].wait()
        for cp in ocopies[c % 2]:
            cp.wait()

        @plsc.parallel_loop(0, CHUNK, unroll=8)
        def body(r):
            ridx = jnp.full((L,), r, dtype=jnp.int32)
            lo = plsc.load_gather(rows_v, [ridx, cols_lo]) * m_lo
            hi = plsc.load_gather(rows_v, [ridx, cols_hi]) * m_hi
            plsc.store_scatter(outc_v, [j_lo, ridx], lo)
            plsc.store_scatter(outc_v, [j_hi, ridx], hi)

        tile_c = wid * NCHUNK + c
        ocopies[c % 2] = [
            pltpu.async_copy(
                outc_v.at[pl.ds(tr * 8, 8), pl.ds(0, CHUNK)],
                out_hbm.at[tr * (BATCH // CHUNK) + tile_c],
                osems[c % 2])
            for tr in range(NBAND)
        ]

    for cps in ocopies:
        for cp in cps:
            cp.wait()


TBLK = 4096
NTBLK = (N_NODES + TBLK - 1) // TBLK


def _transpose_pad_body(i_ref, o_ref):
    o_ref[...] = jnp.pad(i_ref[...].T, ((0, 0), (0, DEGP - DEG)))


_transpose_pad = pl.pallas_call(
    _transpose_pad_body,
    grid=(NTBLK,),
    in_specs=[pl.BlockSpec((DEG, TBLK), lambda i: (0, i))],
    out_specs=pl.BlockSpec((TBLK, DEGP), lambda i: (i, 0)),
    out_shape=jax.ShapeDtypeStruct((N_NODES, DEGP), jnp.float32),
)


def kernel(adj_info, ids, num_samples, layer):
    del layer
    adj_pad = _transpose_pad(adj_info.T)
    adj2 = adj_pad.reshape(2 * N_NODES, DEG)
    cols = jnp.asarray(_PERM[:S])
    mask = (jnp.arange(S) < num_samples).astype(jnp.float32)
    ids2d = ids.reshape(NW * NCHUNK, CHUNK)
    out3 = _sample_kernel(adj2, ids2d, cols, mask)
    out = out3.reshape(NBAND, BATCH // CHUNK, 8, CHUNK)
    return out.transpose(1, 3, 0, 2).reshape(BATCH, S)


# speedup vs baseline: 1.7229x; 1.1291x over previous
import functools

import jax
import jax.numpy as jnp
import numpy as np
from jax import lax
from jax.experimental import pallas as pl
from jax.experimental.pallas import tpu as pltpu
from jax.experimental.pallas import tpu_sc as plsc

N_NODES = 100000
DEG = 64
DEGP = 128
BATCH = 16384
S = 32

_info = plsc.get_sparse_core_info()
NC, NS, L = _info.num_cores, _info.num_subcores, _info.num_lanes
NW = NC * NS
B_PER_W = BATCH // NW
CHUNK = 128
CHUNKP = CHUNK + 1
NCHUNK = B_PER_W // CHUNK
NBAND = S // 8
NBLK = BATCH // CHUNK * NBAND

_PERM = np.array([
    35, 45, 31, 63, 7, 4, 29, 44, 16, 58, 37, 19, 61, 2, 34, 5,
    30, 42, 3, 39, 56, 22, 6, 54, 18, 10, 11, 53, 32, 15, 49, 50,
    20, 43, 8, 24, 9, 40, 59, 25, 13, 52, 62, 60, 47, 33, 14, 17,
    38, 23, 0, 41, 21, 26, 57, 1, 28, 48, 36, 55, 51, 27, 12, 46,
], dtype=np.int32)

_mesh = plsc.VectorSubcoreMesh(core_axis_name="c", subcore_axis_name="s")


@functools.partial(
    pl.kernel,
    mesh=_mesh,
    compiler_params=pltpu.CompilerParams(
        needs_layout_passes=False, use_tc_tiling_on_sc=False),
    out_type=jax.ShapeDtypeStruct((NBLK, 8, CHUNK), jnp.float32),
    scratch_types=[
        pltpu.VMEM((NCHUNK, CHUNK), jnp.int32),
        pltpu.VMEM((NCHUNK, CHUNK), jnp.int32),
        pltpu.VMEM((CHUNK, DEG), jnp.float32),
        pltpu.VMEM((CHUNK, DEG), jnp.float32),
        pltpu.VMEM((CHUNK, DEG), jnp.float32),
        pltpu.VMEM((CHUNK, DEG), jnp.float32),
        pltpu.VMEM((S, CHUNKP), jnp.float32),
        pltpu.VMEM((S, CHUNKP), jnp.float32),
        pltpu.VMEM((S,), jnp.int32),
        pltpu.VMEM((S,), jnp.float32),
        pltpu.SemaphoreType.DMA,
        pltpu.SemaphoreType.DMA,
        pltpu.SemaphoreType.DMA,
        pltpu.SemaphoreType.DMA,
        pltpu.SemaphoreType.DMA,
        pltpu.SemaphoreType.DMA,
    ],
)
def _sample_kernel(adj_hbm, ids_hbm, cols_hbm, mask_hbm, out_hbm,
                   idx_v, qidx_v, rows0_v, rows1_v, rows2_v, rows3_v,
                   outc0_v, outc1_v, cols_v, mask_v,
                   gsem0, gsem1, gsem2, gsem3, osem0, osem1):
    wid = lax.axis_index("s") * NC + lax.axis_index("c")

    pltpu.sync_copy(ids_hbm.at[pl.ds(wid * NCHUNK, NCHUNK)], idx_v)
    pltpu.sync_copy(cols_hbm, cols_v)
    pltpu.sync_copy(mask_hbm, mask_v)
    for c in range(NCHUNK):
        for g in range(CHUNK // L):
            nv = idx_v[c, pl.ds(g * L, L)]
            qidx_v[c, pl.ds(g * L, L)] = lax.shift_left(nv, 1)

    rows_bufs = (rows0_v, rows1_v, rows2_v, rows3_v)
    gsems = (gsem0, gsem1, gsem2, gsem3)
    out_bufs = (outc0_v, outc1_v)
    osems = (osem0, osem1)

    gcopies = [
        pltpu.async_copy(adj_hbm.at[qidx_v.at[c]], rows_bufs[c], gsems[c])
        for c in range(NCHUNK)
    ]

    cols_lo = cols_v[pl.ds(0, L)]
    cols_hi = cols_v[pl.ds(L, L)]
    m_lo = mask_v[pl.ds(0, L)]
    m_hi = mask_v[pl.ds(L, L)]
    j_lo = lax.iota(jnp.int32, L)
    j_hi = j_lo + L

    ocopies = [[], []]
    for c in range(NCHUNK):
        rows_v = rows_bufs[c]
        outc_v = out_bufs[c % 2]
        gcopies[c].wait()
        for cp in ocopies[c % 2]:
            cp.wait()

        @plsc.parallel_loop(0, CHUNK, unroll=8)
        def body(r):
            ridx = jnp.full((L,), r, dtype=jnp.int32)
            lo = plsc.load_gather(rows_v, [ridx, cols_lo]) * m_lo
            hi = plsc.load_gather(rows_v, [ridx, cols_hi]) * m_hi
            plsc.store_scatter(outc_v, [j_lo, ridx], lo)
            plsc.store_scatter(outc_v, [j_hi, ridx], hi)

        tile_c = wid * NCHUNK + c
        ocopies[c % 2] = [
            pltpu.async_copy(
                outc_v.at[pl.ds(tr * 8, 8), pl.ds(0, CHUNK)],
                out_hbm.at[tr * (BATCH // CHUNK) + tile_c],
                osems[c % 2])
            for tr in range(NBAND)
        ]

    for cps in ocopies:
        for cp in cps:
            cp.wait()


TBLK = 8192
NTBLK = (N_NODES + TBLK - 1) // TBLK


def _transpose_pad_body(i_ref, o_ref):
    o_ref[...] = jnp.pad(i_ref[...].T, ((0, 0), (0, DEGP - DEG)))


_transpose_pad = pl.pallas_call(
    _transpose_pad_body,
    grid=(NTBLK,),
    in_specs=[pl.BlockSpec((DEG, TBLK), lambda i: (0, i))],
    out_specs=pl.BlockSpec((TBLK, DEGP), lambda i: (i, 0)),
    out_shape=jax.ShapeDtypeStruct((N_NODES, DEGP), jnp.float32),
)


def kernel(adj_info, ids, num_samples, layer):
    del layer
    adj_pad = _transpose_pad(adj_info.T)
    adj2 = adj_pad.reshape(2 * N_NODES, DEG)
    cols = jnp.asarray(_PERM[:S])
    mask = (jnp.arange(S) < num_samples).astype(jnp.float32)
    ids2d = ids.reshape(NW * NCHUNK, CHUNK)
    out3 = _sample_kernel(adj2, ids2d, cols, mask)
    out = out3.reshape(NBAND, BATCH // CHUNK, 8, CHUNK)
    return out.transpose(1, 3, 0, 2).reshape(BATCH, S)


# speedup vs baseline: 1.7725x; 1.0288x over previous
import functools

import jax
import jax.numpy as jnp
import numpy as np
from jax import lax
from jax.experimental import pallas as pl
from jax.experimental.pallas import tpu as pltpu
from jax.experimental.pallas import tpu_sc as plsc

N_NODES = 100000
DEG = 64
DEGP = 128
BATCH = 16384
S = 32

_info = plsc.get_sparse_core_info()
NC, NS, L = _info.num_cores, _info.num_subcores, _info.num_lanes
NW = NC * NS
B_PER_W = BATCH // NW
CHUNK = 128
CHUNKP = CHUNK + 1
NCHUNK = B_PER_W // CHUNK
NBAND = S // 8
NBLK = BATCH // CHUNK * NBAND

_PERM = np.array([
    35, 45, 31, 63, 7, 4, 29, 44, 16, 58, 37, 19, 61, 2, 34, 5,
    30, 42, 3, 39, 56, 22, 6, 54, 18, 10, 11, 53, 32, 15, 49, 50,
    20, 43, 8, 24, 9, 40, 59, 25, 13, 52, 62, 60, 47, 33, 14, 17,
    38, 23, 0, 41, 21, 26, 57, 1, 28, 48, 36, 55, 51, 27, 12, 46,
], dtype=np.int32)

_mesh = plsc.VectorSubcoreMesh(core_axis_name="c", subcore_axis_name="s")


@functools.partial(
    pl.kernel,
    mesh=_mesh,
    compiler_params=pltpu.CompilerParams(
        needs_layout_passes=False, use_tc_tiling_on_sc=False),
    out_type=jax.ShapeDtypeStruct((NBLK, 8, CHUNK), jnp.float32),
    scratch_types=[
        pltpu.VMEM((NCHUNK, CHUNK), jnp.int32),
        pltpu.VMEM((NCHUNK, CHUNK), jnp.int32),
        pltpu.VMEM((CHUNK, DEG), jnp.float32),
        pltpu.VMEM((CHUNK, DEG), jnp.float32),
        pltpu.VMEM((CHUNK, DEG), jnp.float32),
        pltpu.VMEM((CHUNK, DEG), jnp.float32),
        pltpu.VMEM((S, CHUNKP), jnp.float32),
        pltpu.VMEM((S, CHUNKP), jnp.float32),
        pltpu.VMEM((S,), jnp.int32),
        pltpu.VMEM((S,), jnp.float32),
        pltpu.SemaphoreType.DMA,
        pltpu.SemaphoreType.DMA,
        pltpu.SemaphoreType.DMA,
        pltpu.SemaphoreType.DMA,
        pltpu.SemaphoreType.DMA,
        pltpu.SemaphoreType.DMA,
    ],
)
def _sample_kernel(adj_hbm, ids_hbm, cols_hbm, mask_hbm, out_hbm,
                   idx_v, qidx_v, rows0_v, rows1_v, rows2_v, rows3_v,
                   outc0_v, outc1_v, cols_v, mask_v,
                   gsem0, gsem1, gsem2, gsem3, osem0, osem1):
    wid = lax.axis_index("s") * NC + lax.axis_index("c")

    pltpu.sync_copy(ids_hbm.at[pl.ds(wid * NCHUNK, NCHUNK)], idx_v)
    pltpu.sync_copy(cols_hbm, cols_v)
    pltpu.sync_copy(mask_hbm, mask_v)
    for c in range(NCHUNK):
        for g in range(CHUNK // L):
            nv = idx_v[c, pl.ds(g * L, L)]
            qidx_v[c, pl.ds(g * L, L)] = lax.shift_left(nv, 1)

    rows_bufs = (rows0_v, rows1_v, rows2_v, rows3_v)
    gsems = (gsem0, gsem1, gsem2, gsem3)
    out_bufs = (outc0_v, outc1_v)
    osems = (osem0, osem1)

    gcopies = [
        pltpu.async_copy(adj_hbm.at[qidx_v.at[c]], rows_bufs[c], gsems[c])
        for c in range(NCHUNK)
    ]

    cols_lo = cols_v[pl.ds(0, L)]
    cols_hi = cols_v[pl.ds(L, L)]
    m_lo = mask_v[pl.ds(0, L)]
    m_hi = mask_v[pl.ds(L, L)]
    j_lo = lax.iota(jnp.int32, L)
    j_hi = j_lo + L

    ocopies = [[], []]
    for c in range(NCHUNK):
        rows_v = rows_bufs[c]
        outc_v = out_bufs[c % 2]
        gcopies[c].wait()
        for cp in ocopies[c % 2]:
            cp.wait()

        @plsc.parallel_loop(0, CHUNK, unroll=8)
        def body(r):
            ridx = jnp.full((L,), r, dtype=jnp.int32)
            lo = plsc.load_gather(rows_v, [ridx, cols_lo]) * m_lo
            hi = plsc.load_gather(rows_v, [ridx, cols_hi]) * m_hi
            plsc.store_scatter(outc_v, [j_lo, ridx], lo)
            plsc.store_scatter(outc_v, [j_hi, ridx], hi)

        tile_c = wid * NCHUNK + c
        ocopies[c % 2] = [
            pltpu.async_copy(
                outc_v.at[pl.ds(tr * 8, 8), pl.ds(0, CHUNK)],
                out_hbm.at[tr * (BATCH // CHUNK) + tile_c],
                osems[c % 2])
            for tr in range(NBAND)
        ]

    for cps in ocopies:
        for cp in cps:
            cp.wait()


TBLK = 16384
NTBLK = (N_NODES + TBLK - 1) // TBLK


def _transpose_pad_body(i_ref, o_ref):
    o_ref[...] = jnp.pad(i_ref[...].T, ((0, 0), (0, DEGP - DEG)))


_transpose_pad = pl.pallas_call(
    _transpose_pad_body,
    grid=(NTBLK,),
    in_specs=[pl.BlockSpec((DEG, TBLK), lambda i: (0, i))],
    out_specs=pl.BlockSpec((TBLK, DEGP), lambda i: (i, 0)),
    out_shape=jax.ShapeDtypeStruct((N_NODES, DEGP), jnp.float32),
)


def kernel(adj_info, ids, num_samples, layer):
    del layer
    adj_pad = _transpose_pad(adj_info.T)
    adj2 = adj_pad.reshape(2 * N_NODES, DEG)
    cols = jnp.asarray(_PERM[:S])
    mask = (jnp.arange(S) < num_samples).astype(jnp.float32)
    ids2d = ids.reshape(NW * NCHUNK, CHUNK)
    out3 = _sample_kernel(adj2, ids2d, cols, mask)
    out = out3.reshape(NBAND, BATCH // CHUNK, 8, CHUNK)
    return out.transpose(1, 3, 0, 2).reshape(BATCH, S)


# speedup vs baseline: 1.7898x; 1.0098x over previous
import functools

import jax
import jax.numpy as jnp
import numpy as np
from jax import lax
from jax.experimental import pallas as pl
from jax.experimental.pallas import tpu as pltpu
from jax.experimental.pallas import tpu_sc as plsc

N_NODES = 100000
DEG = 64
DEGP = 128
BATCH = 16384
S = 32

_info = plsc.get_sparse_core_info()
NC, NS, L = _info.num_cores, _info.num_subcores, _info.num_lanes
NW = NC * NS
B_PER_W = BATCH // NW
CHUNK = 128
CHUNKP = CHUNK + 1
NCHUNK = B_PER_W // CHUNK
NBAND = S // 8
NBLK = BATCH // CHUNK * NBAND

_PERM = np.array([
    35, 45, 31, 63, 7, 4, 29, 44, 16, 58, 37, 19, 61, 2, 34, 5,
    30, 42, 3, 39, 56, 22, 6, 54, 18, 10, 11, 53, 32, 15, 49, 50,
    20, 43, 8, 24, 9, 40, 59, 25, 13, 52, 62, 60, 47, 33, 14, 17,
    38, 23, 0, 41, 21, 26, 57, 1, 28, 48, 36, 55, 51, 27, 12, 46,
], dtype=np.int32)

_mesh = plsc.VectorSubcoreMesh(core_axis_name="c", subcore_axis_name="s")


@functools.partial(
    pl.kernel,
    mesh=_mesh,
    compiler_params=pltpu.CompilerParams(
        needs_layout_passes=False, use_tc_tiling_on_sc=False),
    out_type=jax.ShapeDtypeStruct((NBLK, 8, CHUNK), jnp.float32),
    scratch_types=[
        pltpu.VMEM((NCHUNK, CHUNK), jnp.int32),
        pltpu.VMEM((NCHUNK, CHUNK), jnp.int32),
        pltpu.VMEM((CHUNK, DEG), jnp.float32),
        pltpu.VMEM((CHUNK, DEG), jnp.float32),
        pltpu.VMEM((CHUNK, DEG), jnp.float32),
        pltpu.VMEM((CHUNK, DEG), jnp.float32),
        pltpu.VMEM((S, CHUNKP), jnp.float32),
        pltpu.VMEM((S, CHUNKP), jnp.float32),
        pltpu.VMEM((S,), jnp.int32),
        pltpu.VMEM((S,), jnp.float32),
        pltpu.SemaphoreType.DMA,
        pltpu.SemaphoreType.DMA,
        pltpu.SemaphoreType.DMA,
        pltpu.SemaphoreType.DMA,
        pltpu.SemaphoreType.DMA,
        pltpu.SemaphoreType.DMA,
    ],
)
def _sample_kernel(adj_hbm, ids_hbm, cols_hbm, mask_hbm, out_hbm,
                   idx_v, qidx_v, rows0_v, rows1_v, rows2_v, rows3_v,
                   outc0_v, outc1_v, cols_v, mask_v,
                   gsem0, gsem1, gsem2, gsem3, osem0, osem1):
    wid = lax.axis_index("s") * NC + lax.axis_index("c")

    pltpu.sync_copy(ids_hbm.at[pl.ds(wid * NCHUNK, NCHUNK)], idx_v)
    pltpu.sync_copy(cols_hbm, cols_v)
    pltpu.sync_copy(mask_hbm, mask_v)
    for c in range(NCHUNK):
        for g in range(CHUNK // L):
            nv = idx_v[c, pl.ds(g * L, L)]
            qidx_v[c, pl.ds(g * L, L)] = lax.shift_left(nv, 1)

    rows_bufs = (rows0_v, rows1_v, rows2_v, rows3_v)
    gsems = (gsem0, gsem1, gsem2, gsem3)
    out_bufs = (outc0_v, outc1_v)
    osems = (osem0, osem1)

    gcopies = [
        pltpu.async_copy(adj_hbm.at[qidx_v.at[c]], rows_bufs[c], gsems[c])
        for c in range(NCHUNK)
    ]

    cols_lo = cols_v[pl.ds(0, L)]
    cols_hi = cols_v[pl.ds(L, L)]
    m_lo = mask_v[pl.ds(0, L)]
    m_hi = mask_v[pl.ds(L, L)]
    j_lo = lax.iota(jnp.int32, L)
    j_hi = j_lo + L

    ocopies = [[], []]
    for c in range(NCHUNK):
        rows_v = rows_bufs[c]
        outc_v = out_bufs[c % 2]
        gcopies[c].wait()
        for cp in ocopies[c % 2]:
            cp.wait()

        @plsc.parallel_loop(0, CHUNK, unroll=8)
        def body(r):
            ridx = jnp.full((L,), r, dtype=jnp.int32)
            lo = plsc.load_gather(rows_v, [ridx, cols_lo]) * m_lo
            hi = plsc.load_gather(rows_v, [ridx, cols_hi]) * m_hi
            plsc.store_scatter(outc_v, [j_lo, ridx], lo)
            plsc.store_scatter(outc_v, [j_hi, ridx], hi)

        tile_c = wid * NCHUNK + c
        ocopies[c % 2] = [
            pltpu.async_copy(
                outc_v.at[pl.ds(tr * 8, 8), pl.ds(0, CHUNK)],
                out_hbm.at[tr * (BATCH // CHUNK) + tile_c],
                osems[c % 2])
            for tr in range(NBAND)
        ]

    for cps in ocopies:
        for cp in cps:
            cp.wait()


TBLK = 32768
NTBLK = (N_NODES + TBLK - 1) // TBLK


def _transpose_pad_body(i_ref, o_ref):
    o_ref[...] = jnp.pad(i_ref[...].T, ((0, 0), (0, DEGP - DEG)))


_transpose_pad = pl.pallas_call(
    _transpose_pad_body,
    grid=(NTBLK,),
    in_specs=[pl.BlockSpec((DEG, TBLK), lambda i: (0, i))],
    out_specs=pl.BlockSpec((TBLK, DEGP), lambda i: (i, 0)),
    out_shape=jax.ShapeDtypeStruct((N_NODES, DEGP), jnp.float32),
)


def kernel(adj_info, ids, num_samples, layer):
    del layer
    adj_pad = _transpose_pad(adj_info.T)
    adj2 = adj_pad.reshape(2 * N_NODES, DEG)
    cols = jnp.asarray(_PERM[:S])
    mask = (jnp.arange(S) < num_samples).astype(jnp.float32)
    ids2d = ids.reshape(NW * NCHUNK, CHUNK)
    out3 = _sample_kernel(adj2, ids2d, cols, mask)
    out = out3.reshape(NBAND, BATCH // CHUNK, 8, CHUNK)
    return out.transpose(1, 3, 0, 2).reshape(BATCH, S)
